# baseline (speedup 1.0000x reference)
import functools

import jax
import jax.numpy as jnp
from jax import lax
from jax.experimental import pallas as pl
from jax.experimental.pallas import tpu as pltpu
from jax.experimental.pallas import tpu_sc as plsc

RAD = 2
KP = 4096
H = 512
W = 512
B = 8
NW = 32
KPW = (B * KP) // NW
NCHUNK = KPW // 16


def _mp5_cols(x):
    h, w = x.shape
    pad = jnp.full((h, 2), -jnp.inf, x.dtype)
    c = jnp.concatenate([pad, x, pad], axis=1)
    m = c[:, 0:w]
    for i in range(1, 5):
        m = jnp.maximum(m, c[:, i:i + w])
    return m


def _mp5_rows(x):
    h, w = x.shape
    pad = jnp.full((2, w), -jnp.inf, x.dtype)
    c = jnp.concatenate([pad, x, pad], axis=0)
    m = c[0:h]
    for i in range(1, 5):
        m = jnp.maximum(m, c[i:i + h])
    return m


def _mp5(x):
    return _mp5_rows(_mp5_cols(x))


def _nms_body(s_ref, out_ref):
    s = s_ref[0]
    maxm = s == _mp5(s)
    for _ in range(2):
        supp = _mp5(jnp.where(maxm, 1.0, 0.0)) > 0.0
        ss = jnp.where(supp, 0.0, s)
        newm = ss == _mp5(ss)
        maxm = maxm | (newm & (~supp))
    nms = jnp.where(maxm, s, 0.0)
    ri = lax.broadcasted_iota(jnp.int32, (H, W), 0)
    ci = lax.broadcasted_iota(jnp.int32, (H, W), 1)
    interior = (ri >= RAD) & (ri < H - RAD) & (ci >= RAD) & (ci < W - RAD)
    out_ref[0] = jnp.where(interior, nms, 0.0)


def _nms_pallas(s3):
    return pl.pallas_call(
        _nms_body,
        grid=(B,),
        in_specs=[pl.BlockSpec((1, H, W), lambda b: (b, 0, 0))],
        out_specs=pl.BlockSpec((1, H, W), lambda b: (b, 0, 0)),
        out_shape=jax.ShapeDtypeStruct((B, H, W), jnp.float32),
    )(s3)


def _floor_i32(v):
    i = v.astype(jnp.int32)
    return jnp.where(v < i.astype(jnp.float32), i - 1, i)


def _refine_body(idx_hbm, img_hbm, kx_hbm, ky_hbm, dp_hbm, ks_hbm,
                 idxk_v, gi_v, patch_v, bidx_v, bval_v,
                 kx_v, ky_v, dp_v, ks_v, px_v, py_v, sem):
    wid = lax.axis_index("s") * 2 + lax.axis_index("c")
    base = wid * KPW
    img_b = base // KP
    ibase = img_b * (H * W)

    pltpu.sync_copy(idx_hbm.at[pl.ds(base, KPW)], idxk_v)

    def p1(i, _):
        fidx = idxk_v[pl.ds(i * 16, 16)]
        y = lax.shift_right_logical(fidx, 9)
        x = jnp.bitwise_and(fidx, W - 1)
        for p in range(25):
            dy = p // 5 - RAD
            dx = p % 5 - RAD
            yy = y + dy
            xx = x + dx
            valid = (yy >= 0) & (yy < H) & (xx >= 0) & (xx < W)
            yc = jnp.clip(yy, 0, H - 1)
            xc = jnp.clip(xx, 0, W - 1)
            gi = jnp.where(valid, ibase + yc * W + xc, ibase)
            gi_v[pl.ds(p * KPW + i * 16, 16)] = gi
        return 0

    lax.fori_loop(0, NCHUNK, p1, 0)
    pltpu.async_copy(img_hbm.at[gi_v], patch_v, sem).wait()

    def p2(i, _):
        fidx = idxk_v[pl.ds(i * 16, 16)]
        y = lax.shift_right_logical(fidx, 9)
        x = jnp.bitwise_and(fidx, W - 1)
        vals = []
        for p in range(25):
            dy = p // 5 - RAD
            dx = p % 5 - RAD
            yy = y + dy
            xx = x + dx
            valid = (yy >= 0) & (yy < H) & (xx >= 0) & (xx < W)
            v = patch_v[pl.ds(p * KPW + i * 16, 16)]
            vals.append(jnp.where(valid, v, 0.0))
        m = vals[0]
        for p in range(1, 25):
            m = jnp.maximum(m, vals[p])
        s = jnp.zeros((16,), jnp.float32)
        sx = jnp.zeros((16,), jnp.float32)
        sy = jnp.zeros((16,), jnp.float32)
        sq = jnp.zeros((16,), jnp.float32)
        for p in range(25):
            hx = float(p % 5 - RAD)
            hy = float(p // 5 - RAD)
            e = jnp.exp((vals[p] - m) / 0.1)
            s = s + e
            sx = sx + e * hx
            sy = sy + e * hy
            sq = sq + e * (hx * hx + hy * hy)
        xr = sx / s
        yr = sy / s
        dp_v[pl.ds(i * 16, 16)] = (sq / s - (xr * xr + yr * yr)) * 0.25
        xf = x.astype(jnp.float32)
        yf = y.astype(jnp.float32)
        kx = (xf + xr) / (W - 1) * 2.0 - 1.0
        ky = (yf + yr) / (H - 1) * 2.0 - 1.0
        kx_v[pl.ds(i * 16, 16)] = kx
        ky_v[pl.ds(i * 16, 16)] = ky
        px = (kx + 1.0) / 2.0 * (W - 1)
        py = (ky + 1.0) / 2.0 * (H - 1)
        px_v[pl.ds(i * 16, 16)] = px
        py_v[pl.ds(i * 16, 16)] = py
        x0 = _floor_i32(px)
        y0 = _floor_i32(py)
        x0c = jnp.clip(x0, 0, W - 1)
        x1c = jnp.clip(x0 + 1, 0, W - 1)
        y0c = jnp.clip(y0, 0, H - 1)
        y1c = jnp.clip(y0 + 1, 0, H - 1)
        bidx_v[pl.ds(0 * KPW + i * 16, 16)] = ibase + y0c * W + x0c
        bidx_v[pl.ds(1 * KPW + i * 16, 16)] = ibase + y1c * W + x0c
        bidx_v[pl.ds(2 * KPW + i * 16, 16)] = ibase + y0c * W + x1c
        bidx_v[pl.ds(3 * KPW + i * 16, 16)] = ibase + y1c * W + x1c
        return 0

    lax.fori_loop(0, NCHUNK, p2, 0)
    pltpu.async_copy(img_hbm.at[bidx_v], bval_v, sem).wait()

    def p3(i, _):
        px = px_v[pl.ds(i * 16, 16)]
        py = py_v[pl.ds(i * 16, 16)]
        x0f = _floor_i32(px).astype(jnp.float32)
        y0f = _floor_i32(py).astype(jnp.float32)
        wx1 = px - x0f
        wx0 = (x0f + 1.0) - px
        wy1 = py - y0f
        wy0 = (y0f + 1.0) - py
        va = bval_v[pl.ds(0 * KPW + i * 16, 16)]
        vb = bval_v[pl.ds(1 * KPW + i * 16, 16)]
        vc = bval_v[pl.ds(2 * KPW + i * 16, 16)]
        vd = bval_v[pl.ds(3 * KPW + i * 16, 16)]
        ks_v[pl.ds(i * 16, 16)] = (wx0 * wy0 * va + wx0 * wy1 * vb
                                   + wx1 * wy0 * vc + wx1 * wy1 * vd)
        return 0

    lax.fori_loop(0, NCHUNK, p3, 0)

    pltpu.sync_copy(kx_v, kx_hbm.at[pl.ds(base, KPW)])
    pltpu.sync_copy(ky_v, ky_hbm.at[pl.ds(base, KPW)])
    pltpu.sync_copy(dp_v, dp_hbm.at[pl.ds(base, KPW)])
    pltpu.sync_copy(ks_v, ks_hbm.at[pl.ds(base, KPW)])


def _refine_sc(idx_flat, img_flat):
    mesh = plsc.VectorSubcoreMesh(core_axis_name="c", subcore_axis_name="s")
    f32 = jnp.float32
    i32 = jnp.int32
    out_t = [jax.ShapeDtypeStruct((B * KP,), f32) for _ in range(4)]
    scratch = [
        pltpu.VMEM((KPW,), i32),
        pltpu.VMEM((25 * KPW,), i32),
        pltpu.VMEM((25 * KPW,), f32),
        pltpu.VMEM((4 * KPW,), i32),
        pltpu.VMEM((4 * KPW,), f32),
        pltpu.VMEM((KPW,), f32),
        pltpu.VMEM((KPW,), f32),
        pltpu.VMEM((KPW,), f32),
        pltpu.VMEM((KPW,), f32),
        pltpu.VMEM((KPW,), f32),
        pltpu.VMEM((KPW,), f32),
        pltpu.SemaphoreType.DMA,
    ]
    fn = pl.kernel(_refine_body, out_type=out_t, mesh=mesh,
                   scratch_types=scratch)
    return fn(idx_flat, img_flat)


CH = (H * W) // 4
NV = CH // 16
NB = 1024
NBV = NB // 16
WASTE = NW * KP


def _vex(vec, lane):
    sel = lax.iota(jnp.int32, 16) == lane
    return jnp.max(jnp.where(sel, vec, jnp.int32(-2147483648)))


def _sel_body(keys_hbm, candb_hbm, candi_hbm,
              key_v, hist_v, tmp_v, cnt_v, cnt4_v, ckey_v, cidx_v, didx_v,
              shist, scnt):
    c = lax.axis_index("c")
    s = lax.axis_index("s")
    b = c * 4 + s // 4
    q = s % 4
    bi = s // 4
    w = c * 16 + s
    pltpu.sync_copy(keys_hbm.at[pl.ds(b * (H * W) + q * CH, CH)], key_v)

    zeros16 = jnp.zeros((16,), jnp.int32)
    ones16 = jnp.ones((16,), jnp.int32)
    lanes = lax.iota(jnp.int32, 16)

    prefix = jnp.int32(0)
    n_above = jnp.int32(0)
    target = jnp.int32(KP)
    for lvl in range(3):
        shift = 20 - 10 * lvl

        def clr(i, _):
            hist_v[pl.ds(i * 16, 16)] = zeros16
            return 0

        lax.fori_loop(0, NBV, clr, 0)

        if lvl == 0:
            def hst(i, _):
                k16 = key_v[pl.ds(i * 16, 16)]
                bins = lax.shift_right_logical(k16, 20)
                plsc.addupdate_scatter(hist_v, [bins], ones16)
                return 0
        else:
            pref = prefix
            sh = shift

            def hst(i, _):
                k16 = key_v[pl.ds(i * 16, 16)]
                msk = lax.shift_right_logical(k16, sh + 10) == pref
                bins = jnp.bitwise_and(lax.shift_right_logical(k16, sh),
                                       NB - 1)
                plsc.addupdate_scatter(hist_v, [bins], ones16, mask=msk)
                return 0

        lax.fori_loop(0, NV, hst, 0)
        plsc.subcore_barrier()
        pltpu.sync_copy(hist_v, shist.at[pl.ds(s * NB, NB)])
        plsc.subcore_barrier()
        for j in range(4):
            pltpu.sync_copy(shist.at[pl.ds((bi * 4 + j) * NB, NB)], tmp_v)

            @pl.when(jnp.int32(j) != q)
            def _():
                def acc(i, _):
                    hist_v[pl.ds(i * 16, 16)] = (
                        hist_v[pl.ds(i * 16, 16)] + tmp_v[pl.ds(i * 16, 16)])
                    return 0
                lax.fori_loop(0, NBV, acc, 0)

        def scan(i, car):
            run, beta, ngt, found = car
            basev = (NBV - 1 - i) * 16
            chunk = hist_v[pl.ds(basev, 16)]
            rev = lax.rev(chunk, (0,))
            tot = run + plsc.cumsum(rev)
            hit = (tot >= target) & (found == 0)
            npos = jnp.max(plsc.all_reduce_population_count(hit))
            ffs = jnp.max(plsc.all_reduce_ffs(hit))
            new_beta = basev + 15 - ffs
            new_ngt = _vex(tot, ffs) - _vex(rev, ffs)
            got = (npos > 0) & (found == 0)
            beta = jnp.where(got, new_beta, beta)
            ngt = jnp.where(got, new_ngt, ngt)
            found = jnp.where(got, jnp.int32(1), found)
            run = run + jnp.sum(chunk)
            return run, beta, ngt, found

        _, beta_l, ngt_l, _ = lax.fori_loop(
            0, NBV, scan,
            (jnp.int32(0), jnp.int32(0), jnp.int32(0), jnp.int32(0)))
        prefix = jnp.bitwise_or(lax.shift_left(prefix, 10), beta_l)
        n_above = n_above + ngt_l
        target = target - ngt_l

    T = prefix

    def cntp(i, car):
        gt_a, eq_a = car
        k16 = key_v[pl.ds(i * 16, 16)]
        gt_a = gt_a + plsc.all_reduce_population_count(k16 > T)
        eq_a = eq_a + plsc.all_reduce_population_count(k16 == T)
        return gt_a, eq_a

    gt_a, eq_a = lax.fori_loop(0, NV, cntp, (zeros16, zeros16))
    gt_w = jnp.max(gt_a)
    eq_w = jnp.max(eq_a)
    cnt_v[...] = jnp.where(lanes == 0, gt_w,
                           jnp.where(lanes == 1, eq_w, 0))
    plsc.subcore_barrier()
    pltpu.sync_copy(cnt_v, scnt.at[pl.ds(s * 16, 16)])
    plsc.subcore_barrier()
    pltpu.sync_copy(scnt.at[pl.ds(bi * 64, 64)], cnt4_v)
    gt_pre = jnp.int32(0)
    eq_pre = jnp.int32(0)
    n_gt_tot = jnp.int32(0)
    for j in range(4):
        row = cnt4_v[pl.ds(j * 16, 16)]
        gj = _vex(row, jnp.int32(0))
        ej = _vex(row, jnp.int32(1))
        gt_pre = gt_pre + jnp.where(jnp.int32(j) < q, gj, 0)
        eq_pre = eq_pre + jnp.where(jnp.int32(j) < q, ej, 0)
        n_gt_tot = n_gt_tot + gj
    m = jnp.int32(KP) - n_gt_tot
    keep_eq = jnp.clip(m - eq_pre, 0, eq_w)
    out_off = gt_pre + jnp.minimum(eq_pre, m)

    def cmp_(i, car):
        o, eqr = car
        k16 = key_v[pl.ds(i * 16, 16)]
        gt = k16 > T
        eq = k16 == T
        eqc = plsc.cumsum(eq.astype(jnp.int32))
        keep = eq & ((eqr + eqc) <= keep_eq)
        sel = gt | keep
        fidx = q * CH + i * 16 + lanes
        plsc.store_compressed(ckey_v.at[pl.ds(o, 16)], k16, mask=sel)
        plsc.store_compressed(cidx_v.at[pl.ds(o, 16)], fidx, mask=sel)
        o = o + jnp.max(plsc.all_reduce_population_count(sel))
        eqr = eqr + jnp.max(plsc.all_reduce_population_count(eq))
        return o, eqr

    cnt_w, _ = lax.fori_loop(0, NV, cmp_, (jnp.int32(0), jnp.int32(0)))

    def mkd(i, _):
        pos = i * 16 + lanes
        real = pos < cnt_w
        dst = jnp.where(real, b * KP + out_off + pos,
                        B * KP + w * KP + pos)
        didx_v[pl.ds(i * 16, 16)] = dst
        return 0

    lax.fori_loop(0, KP // 16, mkd, 0)
    pltpu.sync_copy(ckey_v.at[pl.ds(0, KP)], candb_hbm.at[didx_v])
    pltpu.sync_copy(cidx_v.at[pl.ds(0, KP)], candi_hbm.at[didx_v])


def _select_sc(keys_flat):
    mesh = plsc.VectorSubcoreMesh(core_axis_name="c", subcore_axis_name="s")
    i32 = jnp.int32
    out_t = [jax.ShapeDtypeStruct((B * KP + WASTE,), i32) for _ in range(2)]
    scratch = [
        pltpu.VMEM((CH,), i32),
        pltpu.VMEM((NB,), i32),
        pltpu.VMEM((NB,), i32),
        pltpu.VMEM((16,), i32),
        pltpu.VMEM((64,), i32),
        pltpu.VMEM((KP + 16,), i32),
        pltpu.VMEM((KP + 16,), i32),
        pltpu.VMEM((KP,), i32),
        pltpu.VMEM_SHARED((16 * NB,), i32),
        pltpu.VMEM_SHARED((256,), i32),
    ]
    fn = pl.kernel(_sel_body, out_type=out_t, mesh=mesh,
                   scratch_types=scratch,
                   compiler_params=pltpu.CompilerParams(
                       needs_layout_passes=False))
    return fn(keys_flat)


@jax.jit
def kernel(scores_map):
    s3 = scores_map[:, 0]
    nms = _nms_pallas(s3)
    keys_flat = lax.bitcast_convert_type(nms, jnp.int32).reshape(-1)
    candb, candi = _select_sc(keys_flat)
    cand_vals = lax.bitcast_convert_type(candb[:B * KP].reshape(B, KP),
                                         jnp.float32)
    cand_idx = candi[:B * KP].reshape(B, KP)
    _, order = lax.top_k(cand_vals, KP)
    idx = jnp.take_along_axis(cand_idx, order, axis=-1)
    idx_flat = idx.reshape(-1).astype(jnp.int32)
    img_flat = s3.reshape(-1)
    kx, ky, dp, ks = _refine_sc(idx_flat, img_flat)
    kpts = jnp.stack([kx, ky], axis=-1).reshape(B, KP, 2)
    disp = dp.reshape(B, KP)
    kptscores = ks.reshape(B, KP)
    return kpts, disp, kptscores
